# baseline (speedup 1.0000x reference)
import functools

import jax
import jax.numpy as jnp
from jax import lax
from jax.experimental import pallas as pl
from jax.experimental.pallas import tpu as pltpu
from jax.experimental.pallas import tpu_sc as plsc

N_FEATURES = 26
VOCAB = 100001
EMBED = 32
BATCH = 4096
SEQ = 20

N = BATCH * SEQ
NC, NS, L = 2, 16, 16
NW = NC * NS
PER_W = N // NW
B = 128
N_CHUNKS = PER_W // B


def _embed_body(feat_hbm, table_hbm, out_hbm, idx_v, gath_v, gsem, wsem):
  wid = lax.axis_index("s") * NC + lax.axis_index("c")
  base_w = wid * PER_W

  def chunk_body(g, carry):
    base = base_w + g * B
    pltpu.sync_copy(feat_hbm.at[:, pl.ds(base, B)], idx_v)

    def field_fire(i, carry):
      off = i * VOCAB

      def vec_body(j, carry):
        sl = pl.ds(j * L, L)
        idx_v[i, sl] = idx_v[i, sl] + off
        return carry

      lax.fori_loop(0, B // L, vec_body, 0, unroll=True)
      pltpu.async_copy(table_hbm.at[idx_v.at[i]], gath_v.at[i], gsem)
      return carry

    lax.fori_loop(0, N_FEATURES, field_fire, 0)

    def field_write(i, carry):
      pltpu.make_async_copy(
          table_hbm.at[idx_v.at[i]], gath_v.at[i], gsem).wait()
      pltpu.async_copy(gath_v.at[i], out_hbm.at[pl.ds(base, B), i], wsem)
      return carry

    lax.fori_loop(0, N_FEATURES, field_write, 0)

    def field_drain(i, carry):
      pltpu.make_async_copy(
          gath_v.at[i], out_hbm.at[pl.ds(base, B), i], wsem).wait()
      return carry

    lax.fori_loop(0, N_FEATURES, field_drain, 0)
    return carry

  lax.fori_loop(0, N_CHUNKS, chunk_body, 0)


@jax.jit
def kernel(features, tables):
  feat = features.reshape(N_FEATURES, N)
  table = tables.reshape(N_FEATURES * VOCAB, EMBED)
  mesh = plsc.VectorSubcoreMesh(core_axis_name="c", subcore_axis_name="s")
  out = pl.kernel(
      _embed_body,
      out_type=jax.ShapeDtypeStruct((N, N_FEATURES, EMBED), jnp.float32),
      mesh=mesh,
      scratch_types=[
          pltpu.VMEM((N_FEATURES, B), jnp.int32),
          pltpu.VMEM((N_FEATURES, B, EMBED), jnp.float32),
          pltpu.SemaphoreType.DMA,
          pltpu.SemaphoreType.DMA,
      ],
      compiler_params=pltpu.CompilerParams(use_tc_tiling_on_sc=False),
  )(feat, table)
  return out.reshape(BATCH, SEQ, N_FEATURES * EMBED)
